# baseline (speedup 1.0000x reference)
import functools

import jax
import jax.numpy as jnp
from jax import lax
from jax.experimental import pallas as pl
from jax.experimental.pallas import tpu as pltpu
from jax.experimental.pallas import tpu_sc as plsc

_COH = 0.1
_LAM = 1.0


def _token_kernel(x_ref, w_ref, ut_ref, us_ref, st_ref,
                  out_ref, l2_ref, wn_ref, *, n_steps):
    T, Dd = x_ref.shape
    Mm = st_ref.shape[0]
    f32 = jnp.float32
    i = pl.program_id(0)

    @pl.when(i == 0)
    def _():
        W = w_ref[...]
        cn = jnp.sqrt(jnp.sum(W * W, axis=0, keepdims=True))
        wn_ref[...] = W / jnp.maximum(cn, 1e-12)

    Wn = wn_ref[...]
    x = x_ref[...]
    R = lax.dot_general(x, Wn, (((1,), (0,)), ((), ())),
                        preferred_element_type=f32)
    St = st_ref[...]
    l2 = lax.dot_general(R * R, St, (((1,), (1,)), ((), ())),
                         preferred_element_type=f32,
                         precision=lax.Precision.HIGHEST)
    l2_ref[...] = l2

    us = us_ref[...]
    sflat = jnp.tanh(us) * _LAM
    anynz = jnp.max(jnp.abs(sflat)) > 0.0

    @pl.when(anynz)
    def _():
        iota = lax.broadcasted_iota(jnp.int32, (T, Mm), 1)
        v1 = jnp.max(l2, axis=1, keepdims=True)
        i1 = jnp.min(jnp.where(l2 == v1, iota, Mm), axis=1, keepdims=True)
        m1 = iota == i1
        l2b = jnp.where(m1, -1.0, l2)
        v2 = jnp.max(l2b, axis=1, keepdims=True)
        i2 = jnp.min(jnp.where(l2b == v2, iota, Mm), axis=1, keepdims=True)
        M2 = (m1 | (iota == i2)).astype(f32)
        Ut = ut_ref[...]
        ones_r = jnp.ones((1, Dd), f32)
        rnsq = lax.dot_general(ones_r, Ut * Ut, (((1,), (1,)), ((), ())),
                               preferred_element_type=f32)
        sigma = sflat / jnp.maximum(jnp.sqrt(rnsq), 1e-12)
        gate = lax.dot_general(M2, St * sigma, (((1,), (0,)), ((), ())),
                               preferred_element_type=f32)
        writes = lax.dot_general(R * gate, Ut, (((1,), (0,)), ((), ())),
                                 preferred_element_type=f32)
        out_ref[...] = x + writes

    @pl.when(jnp.logical_not(anynz))
    def _():
        out_ref[...] = x


def _gram_kernel(w_ref, ur_ref, us_ref, st_ref, e16_ref, vp_ref, up_ref):
    Dd, MB = w_ref.shape
    f32 = jnp.float32
    W = w_ref[...]
    cn = jnp.sqrt(jnp.sum(W * W, axis=0, keepdims=True))
    Wn = W / jnp.maximum(cn, 1e-12)
    Wnh = Wn.astype(jnp.bfloat16)
    Gv = lax.dot_general(Wnh, Wnh, (((0,), (0,)), ((), ())),
                         preferred_element_type=f32)
    eye = (lax.broadcasted_iota(jnp.int32, (MB, MB), 0)
           == lax.broadcasted_iota(jnp.int32, (MB, MB), 1)).astype(f32)
    pen_scale = (Dd ** 0.5) / (MB * MB)
    vp_ref[...] = jnp.sum(jnp.abs(Gv - eye), axis=(0, 1), keepdims=True) * pen_scale

    us = us_ref[...]
    sflat = jnp.tanh(us) * _LAM
    anynz = jnp.max(jnp.abs(sflat)) > 0.0

    @pl.when(anynz)
    def _():
        Ur = ur_ref[...]
        St = st_ref[...]
        e16 = e16_ref[...]
        usq = Ur * Ur
        rnsq_mb = lax.dot_general(
            lax.dot_general(St, usq, (((1,), (0,)), ((), ())),
                            preferred_element_type=f32),
            e16, (((1,), (1,)), ((), ())), preferred_element_type=f32)
        sig_mat = (jnp.tanh(
            lax.dot_general(St * us, e16, (((1,), (1,)), ((), ())))) * _LAM
            / jnp.maximum(jnp.sqrt(rnsq_mb), 1e-12))
        sig_e = lax.dot_general(sig_mat, e16, (((1,), (0,)), ((), ())),
                                preferred_element_type=f32)
        F = lax.dot_general(St, sig_e, (((0,), (0,)), ((), ())),
                            preferred_element_type=f32)
        Ue = Ur * F
        Gu = lax.dot_general(Ue, Ue, (((0,), (0,)), ((), ())),
                             preferred_element_type=f32)
        up_ref[...] = jnp.sum(jnp.abs(Gu - eye), axis=(0, 1),
                              keepdims=True) * pen_scale

    @pl.when(jnp.logical_not(anynz))
    def _():
        up_ref[...] = jnp.sum(eye, axis=(0, 1), keepdims=True) * pen_scale


def _sc_stats(l2_hbm, en_out, sr_out, rel_out, te_out,
              l2_v, en_l, sr_l, acc_v, shared_en, shared_sr, red_v,
              *, n_tokens, n_experts):
    T, Mm = n_tokens, n_experts
    f32 = jnp.float32
    i32 = jnp.int32
    sid = lax.axis_index("s")
    tpw = T // 16
    n_groups = tpw // 16
    inv_n = 1.0 / T

    pltpu.sync_copy(l2_hbm.at[pl.ds(sid * (tpw * Mm), tpw * Mm)], l2_v)

    zeros16 = jnp.zeros((16,), f32)
    for c in range(Mm // 16):
        en_l[pl.ds(16 * c, 16)] = zeros16
        sr_l[pl.ds(16 * c, 16)] = zeros16

    lane = lax.iota(i32, 16)

    def group_body(g, _):
        base = (g * 16 + lane) * Mm

        def e_body(e, carry):
            b1, j1, b2, j2 = carry
            ev = jnp.full((16,), e, i32)
            v = plsc.load_gather(l2_v, [base + ev])
            gt1 = v > b1
            gt2 = jnp.logical_and(jnp.logical_not(gt1), v > b2)
            b2n = jnp.where(gt1, b1, jnp.where(gt2, v, b2))
            j2n = jnp.where(gt1, j1, jnp.where(gt2, ev, j2))
            b1n = jnp.where(gt1, v, b1)
            j1n = jnp.where(gt1, ev, j1)
            return (b1n, j1n, b2n, j2n)

        init = (jnp.full((16,), -1.0, f32), jnp.zeros((16,), i32),
                jnp.full((16,), -1.0, f32), jnp.zeros((16,), i32))
        b1, j1, b2, j2 = lax.fori_loop(0, Mm, e_body, init)
        ones = jnp.full((16,), inv_n, f32)
        plsc.addupdate_scatter(en_l, [j1], b1 * inv_n)
        plsc.addupdate_scatter(en_l, [j2], b2 * inv_n)
        plsc.addupdate_scatter(sr_l, [j1], ones)
        plsc.addupdate_scatter(sr_l, [j2], ones)
        return 0

    lax.fori_loop(0, n_groups, group_body, 0)

    pltpu.sync_copy(en_l, shared_en.at[pl.ds(sid * Mm, Mm)])
    pltpu.sync_copy(sr_l, shared_sr.at[pl.ds(sid * Mm, Mm)])
    plsc.subcore_barrier()

    @pl.when(sid == 0)
    def _():
        pltpu.sync_copy(shared_en, acc_v)
        tot = [jnp.zeros((16,), f32) for _ in range(Mm // 16)]
        for r in range(16):
            for c in range(Mm // 16):
                tot[c] = tot[c] + acc_v[pl.ds(r * Mm + 16 * c, 16)]
        for c in range(Mm // 16):
            red_v[pl.ds(16 * c, 16)] = tot[c]
        pltpu.sync_copy(red_v, en_out)
        s = tot[0]
        for c in range(1, Mm // 16):
            s = s + tot[c]
        te_v = jnp.maximum(jnp.full((16,), jnp.sum(s), f32),
                           jnp.full((16,), 1e-12, f32))
        red_v[pl.ds(0, 16)] = te_v
        pltpu.sync_copy(red_v.at[pl.ds(0, 16)], te_out)
        for c in range(Mm // 16):
            red_v[pl.ds(16 * c, 16)] = tot[c] / te_v
        pltpu.sync_copy(red_v, rel_out)
        pltpu.sync_copy(shared_sr, acc_v)
        tot = [jnp.zeros((16,), f32) for _ in range(Mm // 16)]
        for r in range(16):
            for c in range(Mm // 16):
                tot[c] = tot[c] + acc_v[pl.ds(r * Mm + 16 * c, 16)]
        for c in range(Mm // 16):
            red_v[pl.ds(16 * c, 16)] = tot[c]
        pltpu.sync_copy(red_v, sr_out)


def kernel(x, V, U, u_scales):
    _, T, Dd = x.shape
    Dv, Mm, Bb = V.shape
    MB = Mm * Bb
    f32 = jnp.float32

    x2 = x.reshape(T, Dd)
    W = V.reshape(Dv, MB)
    Ut = U.transpose(0, 2, 1).reshape(MB, Dd)
    Ur = U.reshape(MB, Dd)

    q = jnp.arange(MB, dtype=jnp.int32)
    St = (q[None, :] // Bb == jnp.arange(Mm, dtype=jnp.int32)[:, None]).astype(f32)
    E16 = (q[None, :] % Bb == jnp.arange(Bb, dtype=jnp.int32)[:, None]).astype(f32)
    us_flat = u_scales.reshape(1, MB)

    n_steps = 4
    tile = T // n_steps

    out2, l2 = pl.pallas_call(
        functools.partial(_token_kernel, n_steps=n_steps),
        grid=(n_steps,),
        in_specs=[
            pl.BlockSpec((tile, Dd), lambda i: (i, 0)),
            pl.BlockSpec((Dd, MB), lambda i: (0, 0)),
            pl.BlockSpec((MB, Dd), lambda i: (0, 0)),
            pl.BlockSpec((1, MB), lambda i: (0, 0)),
            pl.BlockSpec((Mm, MB), lambda i: (0, 0)),
        ],
        out_specs=[
            pl.BlockSpec((tile, Dd), lambda i: (i, 0)),
            pl.BlockSpec((tile, Mm), lambda i: (i, 0)),
        ],
        out_shape=(
            jax.ShapeDtypeStruct((T, Dd), f32),
            jax.ShapeDtypeStruct((T, Mm), f32),
        ),
        scratch_shapes=[pltpu.VMEM((Dv, MB), f32)],
    )(x2, W, Ut, us_flat, St)

    mesh = plsc.VectorSubcoreMesh(core_axis_name="c", subcore_axis_name="s",
                                  num_cores=1)
    sc = pl.kernel(
        functools.partial(_sc_stats, n_tokens=T, n_experts=Mm),
        mesh=mesh,
        compiler_params=pltpu.CompilerParams(needs_layout_passes=False),
        out_type=(
            jax.ShapeDtypeStruct((Mm,), f32),
            jax.ShapeDtypeStruct((Mm,), f32),
            jax.ShapeDtypeStruct((Mm,), f32),
            jax.ShapeDtypeStruct((16,), f32),
        ),
        scratch_types=[
            pltpu.VMEM(((T // 16) * Mm,), f32),
            pltpu.VMEM((Mm,), f32),
            pltpu.VMEM((Mm,), f32),
            pltpu.VMEM((16 * Mm,), f32),
            pltpu.VMEM_SHARED((16 * Mm,), f32),
            pltpu.VMEM_SHARED((16 * Mm,), f32),
            pltpu.VMEM((Mm,), f32),
        ],
    )
    en, sr, rel, te16 = sc(l2.reshape(T * Mm))

    vp, up = pl.pallas_call(
        _gram_kernel,
        out_shape=(
            jax.ShapeDtypeStruct((1, 1), f32),
            jax.ShapeDtypeStruct((1, 1), f32),
        ),
    )(W, Ur, us_flat, St, E16)

    te = te16[0].reshape(())
    vp_s = vp.reshape(())
    al = _COH * vp_s - te
    return (out2.reshape(x.shape), te, en, rel, sr, vp_s, up.reshape(()), al)


# speedup vs baseline: 1.2204x; 1.2204x over previous
import functools

import jax
import jax.numpy as jnp
from jax import lax
from jax.experimental import pallas as pl
from jax.experimental.pallas import tpu as pltpu

_COH = 0.1
_LAM = 1.0


def _fused_kernel(x_ref, w_ref, ut_ref, ur_ref, us_ref, st_ref, e16_ref,
                  out_ref, te_ref, en_ref, rel_ref, sr_ref,
                  vp_ref, up_ref, al_ref, wn_ref,
                  *, n_tokens, n_steps):
    T, Dd = x_ref.shape
    MB = w_ref.shape[1]
    Mm = st_ref.shape[0]
    f32 = jnp.float32
    i = pl.program_id(0)

    @pl.when(i == 0)
    def _():
        W = w_ref[...]
        cn = jnp.sqrt(jnp.sum(W * W, axis=0, keepdims=True))
        wn_ref[...] = W / jnp.maximum(cn, 1e-12)

    Wn = wn_ref[...]
    x = x_ref[...]
    R = lax.dot_general(x, Wn, (((1,), (0,)), ((), ())),
                        preferred_element_type=f32)
    St = st_ref[...]
    l2 = lax.dot_general(R * R, St, (((1,), (1,)), ((), ())),
                         preferred_element_type=f32,
                         precision=lax.Precision.HIGHEST)

    iota = lax.broadcasted_iota(jnp.int32, (T, Mm), 1)
    v1 = jnp.max(l2, axis=1, keepdims=True)
    i1 = jnp.min(jnp.where(l2 == v1, iota, Mm), axis=1, keepdims=True)
    m1 = iota == i1
    l2b = jnp.where(m1, -1.0, l2)
    v2 = jnp.max(l2b, axis=1, keepdims=True)
    i2 = jnp.min(jnp.where(l2b == v2, iota, Mm), axis=1, keepdims=True)
    M2 = (m1 | (iota == i2)).astype(f32)

    inv_n = 1.0 / n_tokens
    en_part = jnp.sum(l2 * M2, axis=0, keepdims=True) * inv_n
    sr_part = jnp.sum(M2, axis=0, keepdims=True) * inv_n

    @pl.when(i == 0)
    def _():
        en_ref[...] = en_part
        sr_ref[...] = sr_part

    @pl.when(i > 0)
    def _():
        en_ref[...] += en_part
        sr_ref[...] += sr_part

    us = us_ref[...]
    sflat = jnp.tanh(us) * _LAM
    anynz = jnp.max(jnp.abs(sflat)) > 0.0

    @pl.when(anynz)
    def _():
        Ut = ut_ref[...]
        ones_r = jnp.ones((1, Dd), f32)
        rnsq = lax.dot_general(ones_r, Ut * Ut, (((1,), (1,)), ((), ())),
                               preferred_element_type=f32)
        sigma = sflat / jnp.maximum(jnp.sqrt(rnsq), 1e-12)
        Dsig = St * sigma
        gate = lax.dot_general(M2, Dsig, (((1,), (0,)), ((), ())),
                               preferred_element_type=f32)
        writes = lax.dot_general(R * gate, Ut, (((1,), (0,)), ((), ())),
                                 preferred_element_type=f32)
        out_ref[...] = x + writes

    @pl.when(jnp.logical_not(anynz))
    def _():
        out_ref[...] = x

    @pl.when(i == n_steps - 1)
    def _():
        te = jnp.maximum(jnp.sum(en_ref[...], axis=1, keepdims=True), 1e-12)
        te_ref[...] = te
        rel_ref[...] = en_ref[...] / te

        Wnh = Wn.astype(jnp.bfloat16)
        Gv = lax.dot_general(Wnh, Wnh, (((0,), (0,)), ((), ())),
                             preferred_element_type=f32)
        eye = (lax.broadcasted_iota(jnp.int32, (MB, MB), 0)
               == lax.broadcasted_iota(jnp.int32, (MB, MB), 1)).astype(f32)
        pen_scale = (Dd ** 0.5) / (MB * MB)
        vp = jnp.sum(jnp.abs(Gv - eye), axis=(0, 1), keepdims=True) * pen_scale
        vp_ref[...] = vp
        al_ref[...] = _COH * vp - te

        @pl.when(anynz)
        def _():
            Ur = ur_ref[...]
            e16 = e16_ref[...]
            usq = Ur * Ur
            rnsq_mb = lax.dot_general(
                lax.dot_general(St, usq, (((1,), (0,)), ((), ())),
                                preferred_element_type=f32),
                e16, (((1,), (1,)), ((), ())),
                preferred_element_type=f32)
            sig_mat = (jnp.tanh(
                lax.dot_general(St * us, e16, (((1,), (1,)), ((), ())))) * _LAM
                / jnp.maximum(jnp.sqrt(rnsq_mb), 1e-12))
            sig_e = lax.dot_general(sig_mat, e16, (((1,), (0,)), ((), ())),
                                    preferred_element_type=f32)
            F = lax.dot_general(St, sig_e, (((0,), (0,)), ((), ())),
                                preferred_element_type=f32)
            Ue = Ur * F
            Gu = lax.dot_general(Ue, Ue, (((0,), (0,)), ((), ())),
                                 preferred_element_type=f32)
            up_ref[...] = jnp.sum(jnp.abs(Gu - eye), axis=(0, 1),
                                  keepdims=True) * pen_scale

        @pl.when(jnp.logical_not(anynz))
        def _():
            up_ref[...] = jnp.sum(eye, axis=(0, 1), keepdims=True) * pen_scale


def kernel(x, V, U, u_scales):
    _, T, Dd = x.shape
    Dv, Mm, Bb = V.shape
    MB = Mm * Bb
    f32 = jnp.float32

    x2 = x.reshape(T, Dd)
    W = V.reshape(Dv, MB)
    Ut = U.transpose(0, 2, 1).reshape(MB, Dd)
    Ur = U.reshape(MB, Dd)

    q = jnp.arange(MB, dtype=jnp.int32)
    St = (q[None, :] // Bb == jnp.arange(Mm, dtype=jnp.int32)[:, None]).astype(f32)
    E16 = (q[None, :] % Bb == jnp.arange(Bb, dtype=jnp.int32)[:, None]).astype(f32)
    us_flat = u_scales.reshape(1, MB)

    n_steps = 4
    tile = T // n_steps

    outs = pl.pallas_call(
        functools.partial(_fused_kernel, n_tokens=T, n_steps=n_steps),
        grid=(n_steps,),
        in_specs=[
            pl.BlockSpec((tile, Dd), lambda i: (i, 0)),
            pl.BlockSpec((Dd, MB), lambda i: (0, 0)),
            pl.BlockSpec((MB, Dd), lambda i: (0, 0)),
            pl.BlockSpec((MB, Dd), lambda i: (0, 0)),
            pl.BlockSpec((1, MB), lambda i: (0, 0)),
            pl.BlockSpec((Mm, MB), lambda i: (0, 0)),
            pl.BlockSpec((Bb, MB), lambda i: (0, 0)),
        ],
        out_specs=[
            pl.BlockSpec((tile, Dd), lambda i: (i, 0)),
            pl.BlockSpec((1, 1), lambda i: (0, 0)),
            pl.BlockSpec((1, Mm), lambda i: (0, 0)),
            pl.BlockSpec((1, Mm), lambda i: (0, 0)),
            pl.BlockSpec((1, Mm), lambda i: (0, 0)),
            pl.BlockSpec((1, 1), lambda i: (0, 0)),
            pl.BlockSpec((1, 1), lambda i: (0, 0)),
            pl.BlockSpec((1, 1), lambda i: (0, 0)),
        ],
        out_shape=(
            jax.ShapeDtypeStruct((T, Dd), f32),
            jax.ShapeDtypeStruct((1, 1), f32),
            jax.ShapeDtypeStruct((1, Mm), f32),
            jax.ShapeDtypeStruct((1, Mm), f32),
            jax.ShapeDtypeStruct((1, Mm), f32),
            jax.ShapeDtypeStruct((1, 1), f32),
            jax.ShapeDtypeStruct((1, 1), f32),
            jax.ShapeDtypeStruct((1, 1), f32),
        ),
        scratch_shapes=[pltpu.VMEM((Dv, MB), f32)],
    )(x2, W, Ut, Ur, us_flat, St, E16)
    out2, te, en, rel, sr, vp, up, al = outs
    return (out2.reshape(x.shape), te.reshape(()), en.reshape(Mm),
            rel.reshape(Mm), sr.reshape(Mm), vp.reshape(()), up.reshape(()),
            al.reshape(()))
